# baseline (speedup 1.0000x reference)
import jax
import jax.numpy as jnp
from jax.experimental import pallas as pl
from jax.experimental.pallas import tpu as pltpu

_EPS = 1e-05
_F = 1024
_BLOCK_ROWS = 1024


def _ln_body(x_ref, a_ref, b_ref, o_ref):
    x = x_ref[...]
    mean = jnp.mean(x, axis=-1, keepdims=True)
    xc = x - mean
    var = jnp.mean(xc * xc, axis=-1, keepdims=True)
    inv = jax.lax.rsqrt(var + _EPS)
    o_ref[...] = xc * inv * a_ref[...] + b_ref[...]


def kernel(x, alpha, bias):
    orig_shape = x.shape
    f = orig_shape[-1]
    x2 = x.reshape(-1, f)
    rows = x2.shape[0]
    br = _BLOCK_ROWS
    grid = (rows // br,)

    out = pl.pallas_call(
        _ln_body,
        out_shape=jax.ShapeDtypeStruct((rows, f), x.dtype),
        grid=grid,
        in_specs=[
            pl.BlockSpec((br, f), lambda i: (i, 0)),
            pl.BlockSpec((1, f), lambda i: (0, 0)),
            pl.BlockSpec((1, f), lambda i: (0, 0)),
        ],
        out_specs=pl.BlockSpec((br, f), lambda i: (i, 0)),
        compiler_params=pltpu.CompilerParams(
            dimension_semantics=("parallel",),
            vmem_limit_bytes=60 * 1024 * 1024,
        ),
        name="layer_norm",
    )(x2, alpha.reshape(1, f), bias.reshape(1, f))
    return out.reshape(orig_shape)


# speedup vs baseline: 1.0250x; 1.0250x over previous
import jax
import jax.numpy as jnp
from jax.experimental import pallas as pl
from jax.experimental.pallas import tpu as pltpu

_EPS = 1e-05
_F = 1024
_BLOCK_ROWS = 2048


def _ln_body(x_ref, a_ref, b_ref, o_ref):
    x = x_ref[...]
    mean = jnp.mean(x, axis=-1, keepdims=True)
    xc = x - mean
    var = jnp.mean(xc * xc, axis=-1, keepdims=True)
    inv = jax.lax.rsqrt(var + _EPS)
    o_ref[...] = xc * inv * a_ref[...] + b_ref[...]


def kernel(x, alpha, bias):
    orig_shape = x.shape
    f = orig_shape[-1]
    x2 = x.reshape(-1, f)
    rows = x2.shape[0]
    br = _BLOCK_ROWS
    grid = (rows // br,)

    out = pl.pallas_call(
        _ln_body,
        out_shape=jax.ShapeDtypeStruct((rows, f), x.dtype),
        grid=grid,
        in_specs=[
            pl.BlockSpec((br, f), lambda i: (i, 0)),
            pl.BlockSpec((1, f), lambda i: (0, 0)),
            pl.BlockSpec((1, f), lambda i: (0, 0)),
        ],
        out_specs=pl.BlockSpec((br, f), lambda i: (i, 0)),
        compiler_params=pltpu.CompilerParams(
            dimension_semantics=("parallel",),
            vmem_limit_bytes=60 * 1024 * 1024,
        ),
        name="layer_norm",
    )(x2, alpha.reshape(1, f), bias.reshape(1, f))
    return out.reshape(orig_shape)
